# Initial kernel scaffold; baseline (speedup 1.0000x reference)
import jax
import jax.numpy as jnp
from jax.experimental import pallas as pl


def kernel(x, synapse_weights, neuron_biases, synapse_indices, input_indices, output_indices, non_input_indices):
    raise NotImplementedError("write your pallas kernel here")


# speedup vs baseline: 2.3721x; 2.3721x over previous
import jax
import jax.numpy as jnp
from jax import lax
from jax.experimental import pallas as pl
from jax.experimental.pallas import tpu as pltpu
from jax.experimental.pallas import tpu_sc as plsc

_N = 20
_NPAD = 32
_SPAD = 384
_WTSZ = 1024
_STEPS = 3
_ISZ = _SPAD + _N * 16
_FSZ = _SPAD + 3 * _NPAD


def _tanh(x):
    return 2.0 / (1.0 + jnp.exp(-2.0 * x)) - 1.0


def _brain_body(ibuf_hbm, fbuf_hbm, out_hbm, ibuf_v, fbuf_v, wt_v, v_v):
    @pl.when((lax.axis_index("s") + lax.axis_index("c")) == 0)
    def _():
        pltpu.sync_copy(ibuf_hbm, ibuf_v)
        pltpu.sync_copy(fbuf_hbm, fbuf_v)

        zero = jnp.zeros((16,), jnp.float32)
        for j in range(_WTSZ // 16):
            wt_v[pl.ds(j * 16, 16)] = zero
        for j in range(_SPAD // 16):
            idx = ibuf_v[pl.ds(j * 16, 16)]
            w = fbuf_v[pl.ds(j * 16, 16)]
            plsc.store_scatter(wt_v, [idx], w)

        v_v[pl.ds(0, 16)] = fbuf_v[pl.ds(_SPAD, 16)]
        v_v[pl.ds(16, 16)] = fbuf_v[pl.ds(_SPAD + 16, 16)]
        bias0 = fbuf_v[pl.ds(_SPAD + _NPAD, 16)]
        bias1 = fbuf_v[pl.ds(_SPAD + _NPAD + 16, 16)]
        mask0 = fbuf_v[pl.ds(_SPAD + 2 * _NPAD, 16)]
        mask1 = fbuf_v[pl.ds(_SPAD + 2 * _NPAD + 16, 16)]

        for _ in range(_STEPS):
            nv0 = bias0
            nv1 = bias1
            for s in range(_N):
                bidx = ibuf_v[pl.ds(_SPAD + s * 16, 16)]
                vs = plsc.load_gather(v_v, [bidx])
                nv0 = nv0 + vs * wt_v[pl.ds(s * _NPAD, 16)]
                nv1 = nv1 + vs * wt_v[pl.ds(s * _NPAD + 16, 16)]
            t0 = _tanh(nv0)
            t1 = _tanh(nv1)
            v_v[pl.ds(0, 16)] = nv0 + mask0 * (t0 - nv0)
            v_v[pl.ds(16, 16)] = nv1 + mask1 * (t1 - nv1)

        pltpu.sync_copy(v_v, out_hbm)


def kernel(x, synapse_weights, neuron_biases, synapse_indices, input_indices,
           output_indices, non_input_indices):
    n_syn = synapse_indices.shape[1]
    src = synapse_indices[0]
    dst = synapse_indices[1]
    flat = src * _NPAD + dst
    bidx = jnp.repeat(jnp.arange(_N, dtype=jnp.int32), 16)
    ibuf = jnp.concatenate(
        [flat, jnp.full((_SPAD - n_syn,), _WTSZ - 1, jnp.int32), bidx])
    v0 = jnp.zeros((_NPAD,), jnp.float32).at[input_indices].set(x)
    bias = jnp.zeros((_NPAD,), jnp.float32).at[non_input_indices].set(
        neuron_biases)
    mask = jnp.ones((_NPAD,), jnp.float32).at[output_indices].set(0.0)
    fbuf = jnp.concatenate(
        [synapse_weights, jnp.zeros((_SPAD - n_syn,), jnp.float32),
         v0, bias, mask])

    mesh = plsc.VectorSubcoreMesh(
        core_axis_name="c", subcore_axis_name="s", num_cores=1)
    run = pl.kernel(
        _brain_body,
        mesh=mesh,
        compiler_params=pltpu.CompilerParams(
            needs_layout_passes=False,
            skip_device_barrier=True,
            disable_bounds_checks=True,
            disable_semaphore_checks=True,
        ),
        out_type=jax.ShapeDtypeStruct((_NPAD,), jnp.float32),
        scratch_types=[
            pltpu.VMEM((_ISZ,), jnp.int32),
            pltpu.VMEM((_FSZ,), jnp.float32),
            pltpu.VMEM((_WTSZ,), jnp.float32),
            pltpu.VMEM((_NPAD,), jnp.float32),
        ],
    )
    out = run(ibuf, fbuf)
    return out[output_indices]


# speedup vs baseline: 2.7729x; 1.1690x over previous
import jax
import jax.numpy as jnp
from jax import lax
from jax.experimental import pallas as pl
from jax.experimental.pallas import tpu as pltpu
from jax.experimental.pallas import tpu_sc as plsc

_N = 20
_NPAD = 32
_SPAD = 384
_WTSZ = 1024
_STEPS = 3


def _tanh(x):
    return 2.0 / (1.0 + jnp.exp(-2.0 * x)) - 1.0


def _brain_body(x_hbm, w_hbm, b_hbm, si_hbm, ii_hbm, oi_hbm, ni_hbm, bidx_hbm,
                out_hbm,
                xv, wv, bv, si0, si1, iiv, oiv, niv, bidx_v,
                wt_v, v_v, bias_v, mask_v, sem):
    n_syn = si_hbm.shape[1]

    @pl.when((lax.axis_index("s") + lax.axis_index("c")) == 0)
    def _():
        copies = [
            pltpu.async_copy(x_hbm, xv.at[pl.ds(0, 4)], sem),
            pltpu.async_copy(w_hbm, wv.at[pl.ds(0, n_syn)], sem),
            pltpu.async_copy(b_hbm, bv, sem),
            pltpu.async_copy(si_hbm.at[0], si0.at[pl.ds(0, n_syn)], sem),
            pltpu.async_copy(si_hbm.at[1], si1.at[pl.ds(0, n_syn)], sem),
            pltpu.async_copy(ii_hbm, iiv.at[pl.ds(0, 4)], sem),
            pltpu.async_copy(oi_hbm, oiv.at[pl.ds(0, 2)], sem),
            pltpu.async_copy(ni_hbm, niv, sem),
            pltpu.async_copy(bidx_hbm, bidx_v, sem),
        ]
        zero = jnp.zeros((16,), jnp.float32)
        one = jnp.ones((16,), jnp.float32)
        for j in range(_WTSZ // 16):
            wt_v[pl.ds(j * 16, 16)] = zero
        v_v[pl.ds(0, 16)] = zero
        v_v[pl.ds(16, 16)] = zero
        bias_v[pl.ds(0, 16)] = zero
        bias_v[pl.ds(16, 16)] = zero
        mask_v[pl.ds(0, 16)] = one
        mask_v[pl.ds(16, 16)] = one
        for c in copies:
            c.wait()

        lane = lax.iota(jnp.int32, 16)
        plsc.store_scatter(v_v, [iiv[pl.ds(0, 16)]], xv[pl.ds(0, 16)],
                           mask=lane < 4)
        plsc.store_scatter(bias_v, [niv[pl.ds(0, 16)]], bv[pl.ds(0, 16)])
        plsc.store_scatter(mask_v, [oiv[pl.ds(0, 16)]], zero, mask=lane < 2)

        full_chunks = n_syn // 16
        for j in range(full_chunks + (1 if n_syn % 16 else 0)):
            src = si0[pl.ds(j * 16, 16)]
            dst = si1[pl.ds(j * 16, 16)]
            flat = src * _NPAD + dst
            w = wv[pl.ds(j * 16, 16)]
            m = None if j < full_chunks else lane < (n_syn % 16)
            plsc.store_scatter(wt_v, [flat], w, mask=m)

        bias0 = bias_v[pl.ds(0, 16)]
        bias1 = bias_v[pl.ds(16, 16)]
        mask0 = mask_v[pl.ds(0, 16)]
        mask1 = mask_v[pl.ds(16, 16)]

        for _ in range(_STEPS):
            nv0 = bias0
            nv1 = bias1
            for s in range(_N):
                bidx = bidx_v[pl.ds(s * 16, 16)]
                vs = plsc.load_gather(v_v, [bidx])
                nv0 = nv0 + vs * wt_v[pl.ds(s * _NPAD, 16)]
                nv1 = nv1 + vs * wt_v[pl.ds(s * _NPAD + 16, 16)]
            t0 = _tanh(nv0)
            t1 = _tanh(nv1)
            v_v[pl.ds(0, 16)] = nv0 + mask0 * (t0 - nv0)
            v_v[pl.ds(16, 16)] = nv1 + mask1 * (t1 - nv1)

        pltpu.sync_copy(v_v, out_hbm)


def kernel(x, synapse_weights, neuron_biases, synapse_indices, input_indices,
           output_indices, non_input_indices):
    bidx = jnp.repeat(jnp.arange(_N, dtype=jnp.int32), 16)

    mesh = plsc.VectorSubcoreMesh(
        core_axis_name="c", subcore_axis_name="s", num_cores=1)
    run = pl.kernel(
        _brain_body,
        mesh=mesh,
        compiler_params=pltpu.CompilerParams(
            needs_layout_passes=False,
            use_tc_tiling_on_sc=False,
            skip_device_barrier=True,
            disable_bounds_checks=True,
            disable_semaphore_checks=True,
        ),
        out_type=jax.ShapeDtypeStruct((_NPAD,), jnp.float32),
        scratch_types=[
            pltpu.VMEM((16,), jnp.float32),
            pltpu.VMEM((_SPAD,), jnp.float32),
            pltpu.VMEM((16,), jnp.float32),
            pltpu.VMEM((_SPAD,), jnp.int32),
            pltpu.VMEM((_SPAD,), jnp.int32),
            pltpu.VMEM((16,), jnp.int32),
            pltpu.VMEM((16,), jnp.int32),
            pltpu.VMEM((16,), jnp.int32),
            pltpu.VMEM((_N * 16,), jnp.int32),
            pltpu.VMEM((_WTSZ,), jnp.float32),
            pltpu.VMEM((_NPAD,), jnp.float32),
            pltpu.VMEM((_NPAD,), jnp.float32),
            pltpu.VMEM((_NPAD,), jnp.float32),
            pltpu.SemaphoreType.DMA,
        ],
    )
    out = run(x, synapse_weights, neuron_biases, synapse_indices,
              input_indices, output_indices, non_input_indices, bidx)
    return out[output_indices]


# speedup vs baseline: 2.9733x; 1.0723x over previous
import jax
import jax.numpy as jnp
from jax import lax
from jax.experimental import pallas as pl
from jax.experimental.pallas import tpu as pltpu
from jax.experimental.pallas import tpu_sc as plsc

_N = 20
_NPAD = 32
_SPAD = 384
_WTSZ = 1024
_STEPS = 3


def _tanh(x):
    return 2.0 / (1.0 + jnp.exp(-2.0 * x)) - 1.0


def _brain_body(x_hbm, w_hbm, b_hbm, si_hbm, ii_hbm, oi_hbm, ni_hbm, bidx_hbm,
                out_hbm,
                xv, wv, bv, si0, si1, iiv, oiv, niv, bidx_v,
                wt_v, v_v, bias_v, mask_v, ov, sem):
    n_syn = si_hbm.shape[1]

    @pl.when((lax.axis_index("s") + lax.axis_index("c")) == 0)
    def _():
        copies = [
            pltpu.async_copy(x_hbm, xv.at[pl.ds(0, 4)], sem),
            pltpu.async_copy(w_hbm, wv.at[pl.ds(0, n_syn)], sem),
            pltpu.async_copy(b_hbm, bv, sem),
            pltpu.async_copy(si_hbm.at[0], si0.at[pl.ds(0, n_syn)], sem),
            pltpu.async_copy(si_hbm.at[1], si1.at[pl.ds(0, n_syn)], sem),
            pltpu.async_copy(ii_hbm, iiv.at[pl.ds(0, 4)], sem),
            pltpu.async_copy(oi_hbm, oiv.at[pl.ds(0, 2)], sem),
            pltpu.async_copy(ni_hbm, niv, sem),
            pltpu.async_copy(bidx_hbm, bidx_v, sem),
        ]
        zero = jnp.zeros((16,), jnp.float32)
        one = jnp.ones((16,), jnp.float32)
        for j in range(_WTSZ // 16):
            wt_v[pl.ds(j * 16, 16)] = zero
        v_v[pl.ds(0, 16)] = zero
        v_v[pl.ds(16, 16)] = zero
        bias_v[pl.ds(0, 16)] = zero
        bias_v[pl.ds(16, 16)] = zero
        mask_v[pl.ds(0, 16)] = one
        mask_v[pl.ds(16, 16)] = one
        for c in copies:
            c.wait()

        lane = lax.iota(jnp.int32, 16)
        plsc.store_scatter(v_v, [iiv[pl.ds(0, 16)]], xv[pl.ds(0, 16)],
                           mask=lane < 4)
        plsc.store_scatter(bias_v, [niv[pl.ds(0, 16)]], bv[pl.ds(0, 16)])
        plsc.store_scatter(mask_v, [oiv[pl.ds(0, 16)]], zero, mask=lane < 2)

        full_chunks = n_syn // 16
        for j in range(full_chunks + (1 if n_syn % 16 else 0)):
            src = si0[pl.ds(j * 16, 16)]
            dst = si1[pl.ds(j * 16, 16)]
            flat = src * _NPAD + dst
            w = wv[pl.ds(j * 16, 16)]
            m = None if j < full_chunks else lane < (n_syn % 16)
            plsc.store_scatter(wt_v, [flat], w, mask=m)

        bias0 = bias_v[pl.ds(0, 16)]
        bias1 = bias_v[pl.ds(16, 16)]
        mask0 = mask_v[pl.ds(0, 16)]
        mask1 = mask_v[pl.ds(16, 16)]

        for _ in range(_STEPS):
            nv0 = bias0
            nv1 = bias1
            for s in range(_N):
                bidx = bidx_v[pl.ds(s * 16, 16)]
                vs = plsc.load_gather(v_v, [bidx])
                nv0 = nv0 + vs * wt_v[pl.ds(s * _NPAD, 16)]
                nv1 = nv1 + vs * wt_v[pl.ds(s * _NPAD + 16, 16)]
            t0 = _tanh(nv0)
            t1 = _tanh(nv1)
            v_v[pl.ds(0, 16)] = nv0 + mask0 * (t0 - nv0)
            v_v[pl.ds(16, 16)] = nv1 + mask1 * (t1 - nv1)

        ov[pl.ds(0, 16)] = plsc.load_gather(v_v, [oiv[pl.ds(0, 16)]],
                                            mask=lane < 2)
        pltpu.sync_copy(ov.at[pl.ds(0, 2)], out_hbm)


def kernel(x, synapse_weights, neuron_biases, synapse_indices, input_indices,
           output_indices, non_input_indices):
    bidx = jnp.repeat(jnp.arange(_N, dtype=jnp.int32), 16)

    mesh = plsc.VectorSubcoreMesh(
        core_axis_name="c", subcore_axis_name="s", num_cores=1)
    run = pl.kernel(
        _brain_body,
        mesh=mesh,
        compiler_params=pltpu.CompilerParams(
            needs_layout_passes=False,
            use_tc_tiling_on_sc=False,
            skip_device_barrier=True,
            disable_bounds_checks=True,
            disable_semaphore_checks=True,
        ),
        out_type=jax.ShapeDtypeStruct((2,), jnp.float32),
        scratch_types=[
            pltpu.VMEM((16,), jnp.float32),
            pltpu.VMEM((_SPAD,), jnp.float32),
            pltpu.VMEM((16,), jnp.float32),
            pltpu.VMEM((_SPAD,), jnp.int32),
            pltpu.VMEM((_SPAD,), jnp.int32),
            pltpu.VMEM((16,), jnp.int32),
            pltpu.VMEM((16,), jnp.int32),
            pltpu.VMEM((16,), jnp.int32),
            pltpu.VMEM((_N * 16,), jnp.int32),
            pltpu.VMEM((_WTSZ,), jnp.float32),
            pltpu.VMEM((_NPAD,), jnp.float32),
            pltpu.VMEM((_NPAD,), jnp.float32),
            pltpu.VMEM((_NPAD,), jnp.float32),
            pltpu.VMEM((16,), jnp.float32),
            pltpu.SemaphoreType.DMA,
        ],
    )
    return run(x, synapse_weights, neuron_biases, synapse_indices,
               input_indices, output_indices, non_input_indices, bidx)


# speedup vs baseline: 2.9734x; 1.0000x over previous
import jax
import jax.numpy as jnp
from jax import lax
from jax.experimental import pallas as pl
from jax.experimental.pallas import tpu as pltpu
from jax.experimental.pallas import tpu_sc as plsc

_N = 20
_NPAD = 32
_SPAD = 384
_WTSZ = 1024
_STEPS = 3


def _tanh(x):
    return 2.0 / (1.0 + jnp.exp(-2.0 * x)) - 1.0


def _brain_body(x_hbm, w_hbm, b_hbm, si_hbm, ii_hbm, oi_hbm, ni_hbm, bidx_hbm,
                out_hbm,
                xv, wv, bv, si0, si1, iiv, oiv, niv, bidx_v,
                wt_v, v_v, bias_v, mask_v, ov, sem):
    n_syn = si_hbm.shape[1]

    if True:
        copies = [
            pltpu.async_copy(x_hbm, xv.at[pl.ds(0, 4)], sem),
            pltpu.async_copy(w_hbm, wv.at[pl.ds(0, n_syn)], sem),
            pltpu.async_copy(b_hbm, bv, sem),
            pltpu.async_copy(si_hbm.at[0], si0.at[pl.ds(0, n_syn)], sem),
            pltpu.async_copy(si_hbm.at[1], si1.at[pl.ds(0, n_syn)], sem),
            pltpu.async_copy(ii_hbm, iiv.at[pl.ds(0, 4)], sem),
            pltpu.async_copy(oi_hbm, oiv.at[pl.ds(0, 2)], sem),
            pltpu.async_copy(ni_hbm, niv, sem),
            pltpu.async_copy(bidx_hbm, bidx_v, sem),
        ]
        zero = jnp.zeros((16,), jnp.float32)
        one = jnp.ones((16,), jnp.float32)
        for j in range(_N * _NPAD // 16):
            wt_v[pl.ds(j * 16, 16)] = zero
        v_v[pl.ds(0, 16)] = zero
        v_v[pl.ds(16, 16)] = zero
        bias_v[pl.ds(0, 16)] = zero
        bias_v[pl.ds(16, 16)] = zero
        mask_v[pl.ds(0, 16)] = one
        mask_v[pl.ds(16, 16)] = one
        for c in copies:
            c.wait()

        lane = lax.iota(jnp.int32, 16)
        plsc.store_scatter(v_v, [iiv[pl.ds(0, 16)]], xv[pl.ds(0, 16)],
                           mask=lane < 4)
        plsc.store_scatter(bias_v, [niv[pl.ds(0, 16)]], bv[pl.ds(0, 16)])
        plsc.store_scatter(mask_v, [oiv[pl.ds(0, 16)]], zero, mask=lane < 2)

        full_chunks = n_syn // 16
        for j in range(full_chunks + (1 if n_syn % 16 else 0)):
            src = si0[pl.ds(j * 16, 16)]
            dst = si1[pl.ds(j * 16, 16)]
            flat = src * _NPAD + dst
            w = wv[pl.ds(j * 16, 16)]
            m = None if j < full_chunks else lane < (n_syn % 16)
            plsc.store_scatter(wt_v, [flat], w, mask=m)

        bias0 = bias_v[pl.ds(0, 16)]
        bias1 = bias_v[pl.ds(16, 16)]
        mask0 = mask_v[pl.ds(0, 16)]
        mask1 = mask_v[pl.ds(16, 16)]

        for _ in range(_STEPS):
            nv0 = bias0
            nv1 = bias1
            for s in range(_N):
                bidx = bidx_v[pl.ds(s * 16, 16)]
                vs = plsc.load_gather(v_v, [bidx])
                nv0 = nv0 + vs * wt_v[pl.ds(s * _NPAD, 16)]
                nv1 = nv1 + vs * wt_v[pl.ds(s * _NPAD + 16, 16)]
            t0 = _tanh(nv0)
            t1 = _tanh(nv1)
            v_v[pl.ds(0, 16)] = nv0 + mask0 * (t0 - nv0)
            v_v[pl.ds(16, 16)] = nv1 + mask1 * (t1 - nv1)

        ov[pl.ds(0, 16)] = plsc.load_gather(v_v, [oiv[pl.ds(0, 16)]],
                                            mask=lane < 2)
        pltpu.sync_copy(ov.at[pl.ds(0, 2)], out_hbm)


def kernel(x, synapse_weights, neuron_biases, synapse_indices, input_indices,
           output_indices, non_input_indices):
    bidx = jnp.repeat(jnp.arange(_N, dtype=jnp.int32), 16)

    mesh = plsc.VectorSubcoreMesh(
        core_axis_name="c", subcore_axis_name="s", num_cores=1,
        num_subcores=1)
    run = pl.kernel(
        _brain_body,
        mesh=mesh,
        compiler_params=pltpu.CompilerParams(
            needs_layout_passes=False,
            use_tc_tiling_on_sc=False,
            skip_device_barrier=True,
            disable_bounds_checks=True,
            disable_semaphore_checks=True,
        ),
        out_type=jax.ShapeDtypeStruct((2,), jnp.float32),
        scratch_types=[
            pltpu.VMEM((16,), jnp.float32),
            pltpu.VMEM((_SPAD,), jnp.float32),
            pltpu.VMEM((16,), jnp.float32),
            pltpu.VMEM((_SPAD,), jnp.int32),
            pltpu.VMEM((_SPAD,), jnp.int32),
            pltpu.VMEM((16,), jnp.int32),
            pltpu.VMEM((16,), jnp.int32),
            pltpu.VMEM((16,), jnp.int32),
            pltpu.VMEM((_N * 16,), jnp.int32),
            pltpu.VMEM((_WTSZ,), jnp.float32),
            pltpu.VMEM((_NPAD,), jnp.float32),
            pltpu.VMEM((_NPAD,), jnp.float32),
            pltpu.VMEM((_NPAD,), jnp.float32),
            pltpu.VMEM((16,), jnp.float32),
            pltpu.SemaphoreType.DMA,
        ],
    )
    return run(x, synapse_weights, neuron_biases, synapse_indices,
               input_indices, output_indices, non_input_indices, bidx)
